# baseline (speedup 1.0000x reference)
import jax
import jax.numpy as jnp
from jax.experimental import pallas as pl
from jax.experimental.pallas import tpu as pltpu

NUM_POINT = 128
INIT_STRIDE = 10.0
COARSE_STRIDE = 4.0
DOWN_SAMPLE = 4.0


def _linears_body(fp_ref, wp_ref, wf_ref, b_ref, out_ref):
    a = fp_ref[...]
    w = wp_ref[...]
    pf = jax.lax.dot_general(
        a, w, (((1,), (1,)), ((), ())), preferred_element_type=jnp.float32)
    pf = pf.astype(jnp.bfloat16)
    out = jax.lax.dot_general(
        pf, wf_ref[...], (((1,), (1,)), ((), ())),
        preferred_element_type=jnp.float32)
    out_ref[...] = out + b_ref[...]


def _linears(fp, trans_poly_w, trans_fuse_w, trans_fuse_b):
    n, ktot = fp.shape
    fp16 = fp.astype(jnp.bfloat16)
    wp16 = trans_poly_w.astype(jnp.bfloat16)
    wf16 = trans_fuse_w.astype(jnp.bfloat16)
    out = pl.pallas_call(
        _linears_body,
        out_shape=jax.ShapeDtypeStruct((n, 256), jnp.float32),
    )(fp16, wp16, wf16, trans_fuse_b[None, :])
    return out


def _conv2d(x, w, b, padding):
    out = jax.lax.conv_general_dilated(
        x, w, (1, 1), padding, dimension_numbers=('NCHW', 'OIHW', 'NCHW'))
    return out + b[None, :, None, None]


def _grid_sample(feature, img_idx, pts, h, w):
    ix = ((pts[..., 0] + 1.0) * w - 1.0) / 2.0
    iy = ((pts[..., 1] + 1.0) * h - 1.0) / 2.0
    x0 = jnp.floor(ix)
    y0 = jnp.floor(iy)
    x1 = x0 + 1.0
    y1 = y0 + 1.0
    wx1 = ix - x0
    wx0 = 1.0 - wx1
    wy1 = iy - y0
    wy0 = 1.0 - wy1
    img = img_idx[:, None]

    def tap(xi, yi):
        valid = ((xi >= 0) & (xi < w) & (yi >= 0) & (yi < h)).astype(feature.dtype)
        xc = jnp.clip(xi, 0, w - 1).astype(jnp.int32)
        yc = jnp.clip(yi, 0, h - 1).astype(jnp.int32)
        v = feature[img, :, yc, xc]
        return v * valid[..., None]

    out = (tap(x0, y0) * (wx0 * wy0)[..., None]
           + tap(x1, y0) * (wx1 * wy0)[..., None]
           + tap(x0, y1) * (wx0 * wy1)[..., None]
           + tap(x1, y1) * (wx1 * wy1)[..., None])
    return jnp.transpose(out, (0, 2, 1))


def kernel(cnn_feature, wh, ct_01, ct_ind, ct_img_idx, conv1_w, conv1_b,
           conv2_w, conv2_b, trans_poly_w, trans_fuse_w, trans_fuse_b):
    batch, _, height, width = cnn_feature.shape
    mask = ct_01.reshape(-1)
    ct_ind_f = jnp.where(mask, ct_ind.reshape(-1), 0)
    ct_img_idx_f = jnp.where(mask, ct_img_idx.reshape(-1), 0)
    ct_x = jnp.clip(ct_ind_f % width, 0, width - 1)
    ct_y = jnp.clip(ct_ind_f // width, 0, height - 1)
    ct_offset = wh[ct_img_idx_f, :, ct_y, ct_x].reshape(ct_x.shape[0], -1, 2)
    ct = jnp.stack([ct_x.astype(jnp.float32), ct_y.astype(jnp.float32)], axis=1)
    init_polys = ct_offset * INIT_STRIDE + ct[:, None, :]

    feat = _conv2d(cnn_feature, conv1_w, conv1_b, 'SAME')
    feat = jax.nn.relu(feat)
    feat = _conv2d(feat, conv2_w, conv2_b, 'SAME')
    points = jnp.concatenate([ct[:, None, :], init_polys], axis=1)
    pts_norm = jnp.stack([points[..., 0] / (width / 2.0) - 1.0,
                          points[..., 1] / (height / 2.0) - 1.0], axis=-1)
    feature_points = _grid_sample(feat, ct_img_idx_f, pts_norm, height, width)
    poly_num = init_polys.shape[0]
    fp = feature_points.reshape(poly_num, -1)
    offsets = _linears(fp, trans_poly_w, trans_fuse_w, trans_fuse_b)
    offsets = offsets.reshape(poly_num, NUM_POINT, 2)
    coarse_polys = offsets * COARSE_STRIDE + init_polys
    return init_polys * DOWN_SAMPLE, coarse_polys * DOWN_SAMPLE, ct


# speedup vs baseline: 1.0403x; 1.0403x over previous
import functools

import jax
import jax.numpy as jnp
from jax import lax
from jax.experimental import pallas as pl
from jax.experimental.pallas import tpu as pltpu
from jax.experimental.pallas import tpu_sc as plsc

NUM_POINT = 128
INIT_STRIDE = 10.0
COARSE_STRIDE = 4.0
DOWN_SAMPLE = 4.0

B, C, H, W = 4, 64, 128, 128
MAXOBJ = 128
N = B * MAXOBJ
P1 = NUM_POINT + 1
NPTS = N * P1
NWORK = 32
WH_PER_W = N * NUM_POINT * 2 // NWORK // 128
GS_PER_W = NPTS // NWORK
GS_CHUNKS = GS_PER_W // 16

def _wh_gather_body(wh_hbm, idx_hbm, out_hbm, idx_v, val_v, sem):
    wid = lax.axis_index("s") * 2 + lax.axis_index("c")
    base = wid * WH_PER_W
    pltpu.sync_copy(idx_hbm.at[pl.ds(base, WH_PER_W)], idx_v)
    descs = []
    for j in range(WH_PER_W):
        descs.append(pltpu.async_copy(wh_hbm.at[idx_v.at[j]], val_v.at[j], sem))
    for d in descs:
        d.wait()
    pltpu.sync_copy(val_v, out_hbm.at[pl.ds(base, WH_PER_W)])


def _wh_gather_sc(wh_flat, whidx):
    k = pl.kernel(
        _wh_gather_body,
        out_type=jax.ShapeDtypeStruct((N * 2 * NUM_POINT // 128, 128), jnp.float32),
        mesh=plsc.VectorSubcoreMesh(core_axis_name="c", subcore_axis_name="s"),
        scratch_types=[
            pltpu.VMEM((WH_PER_W, 128), jnp.int32),
            pltpu.VMEM((WH_PER_W, 128), jnp.float32),
            pltpu.SemaphoreType.DMA,
        ],
    )
    return k(wh_flat, whidx)


def _grid_sample_body(feat_hbm, px_hbm, py_hbm, ib_hbm, out_hbm,
                      px_v, py_v, ib_v, idx_v, tap_v, out_v, sem):
    wid = lax.axis_index("s") * 2 + lax.axis_index("c")
    base = wid * GS_PER_W
    pltpu.sync_copy(px_hbm.at[pl.ds(base, GS_PER_W)], px_v)
    pltpu.sync_copy(py_hbm.at[pl.ds(base, GS_PER_W)], py_v)
    pltpu.sync_copy(ib_hbm.at[pl.ds(base, GS_PER_W)], ib_v)

    def chunk(ci, carry):
        off = ci * 16
        px = px_v[pl.ds(off, 16)]
        py = py_v[pl.ds(off, 16)]
        ib = ib_v[pl.ds(off, 16)]
        ix = px - 0.5
        iy = py - 0.5
        xt = ix.astype(jnp.int32)
        yt = iy.astype(jnp.int32)
        x0 = jnp.where(ix < xt.astype(jnp.float32), xt - 1, xt)
        y0 = jnp.where(iy < yt.astype(jnp.float32), yt - 1, yt)
        wx1 = ix - x0.astype(jnp.float32)
        wy1 = iy - y0.astype(jnp.float32)
        wx0 = 1.0 - wx1
        wy0 = 1.0 - wy1
        zero16 = jnp.zeros((16,), jnp.float32)
        wts = []
        for t, (dx, dy, wx, wy) in enumerate(
                ((0, 0, wx0, wy0), (1, 0, wx1, wy0),
                 (0, 1, wx0, wy1), (1, 1, wx1, wy1))):
            xi = x0 + dx
            yi = y0 + dy
            ok = (xi >= 0) & (xi < W) & (yi >= 0) & (yi < H)
            xc = jnp.minimum(jnp.maximum(xi, 0), W - 1)
            yc = jnp.minimum(jnp.maximum(yi, 0), H - 1)
            idx_v[t] = ib + yc * W + xc
            wts.append(jnp.where(ok, wx * wy, zero16))
        descs = []
        for t in range(4):
            descs.append(pltpu.async_copy(feat_hbm.at[idx_v.at[t]],
                                          tap_v.at[t], sem))
        for d in descs:
            d.wait()
        ilv = plsc.PackFormat.INTERLEAVED
        for p in range(16):
            acc = [jnp.zeros((16,), jnp.float32) for _ in range(4)]
            for t in range(4):
                ws = wts[t][p]
                r0 = plsc.bitcast(tap_v[t, p, pl.ds(0, 16)], jnp.bfloat16)
                r1 = plsc.bitcast(tap_v[t, p, pl.ds(16, 16)], jnp.bfloat16)
                e0, o0 = plsc.unpack(r0, format=ilv)
                e1, o1 = plsc.unpack(r1, format=ilv)
                acc[0] = acc[0] + ws * e0
                acc[1] = acc[1] + ws * o0
                acc[2] = acc[2] + ws * e1
                acc[3] = acc[3] + ws * o1
            out_v[p, pl.ds(0, 32)] = plsc.pack(acc[0], acc[1], format=ilv)
            out_v[p, pl.ds(32, 32)] = plsc.pack(acc[2], acc[3], format=ilv)
        pltpu.sync_copy(out_v, out_hbm.at[pl.ds(base + off, 16)])
        return carry

    lax.fori_loop(0, GS_CHUNKS, chunk, 0)


def _grid_sample_sc(feat_rows, px, py, ib):
    k = pl.kernel(
        _grid_sample_body,
        out_type=jax.ShapeDtypeStruct((NPTS, C), jnp.bfloat16),
        mesh=plsc.VectorSubcoreMesh(core_axis_name="c", subcore_axis_name="s"),
        scratch_types=[
            pltpu.VMEM((GS_PER_W,), jnp.float32),
            pltpu.VMEM((GS_PER_W,), jnp.float32),
            pltpu.VMEM((GS_PER_W,), jnp.int32),
            pltpu.VMEM((4, 16), jnp.int32),
            pltpu.VMEM((4, 16, C // 2), jnp.int32),
            pltpu.VMEM((16, C), jnp.bfloat16),
            pltpu.SemaphoreType.DMA,
        ],
        compiler_params=pltpu.CompilerParams(needs_layout_passes=False,
                                             use_tc_tiling_on_sc=False),
    )
    return k(feat_rows, px, py, ib)


def _conv_body(x_ref, w1_ref, b1_ref, w2_ref, b2_ref, out_ref, pad_ref):
    pad_ref[...] = jnp.zeros_like(pad_ref)
    pad_ref[1:H + 1, 1:W + 1, :] = x_ref[0].astype(jnp.bfloat16)
    for rb in range(8):
        r0 = rb * 16
        taps = []
        for dy in range(3):
            for dx in range(3):
                taps.append(
                    pad_ref[r0 + dy:r0 + dy + 16, dx:dx + W, :].reshape(16 * W, C))
        a = jnp.concatenate(taps, axis=1)
        acc = jnp.dot(a, w1_ref[...], preferred_element_type=jnp.float32)
        acc = jnp.maximum(acc + b1_ref[...], 0.0).astype(jnp.bfloat16)
        o = jnp.dot(acc, w2_ref[...], preferred_element_type=jnp.float32)
        o = o + b2_ref[...]
        out_ref[0, r0:r0 + 16] = o.reshape(16, W, C).astype(jnp.bfloat16)


def _conv_refine(x_nhwc, w1cat, b1, w2t, b2):
    return pl.pallas_call(
        _conv_body,
        grid=(B,),
        in_specs=[
            pl.BlockSpec((1, H, W, C), lambda b: (b, 0, 0, 0)),
            pl.BlockSpec((576, 256), lambda b: (0, 0)),
            pl.BlockSpec((1, 256), lambda b: (0, 0)),
            pl.BlockSpec((256, C), lambda b: (0, 0)),
            pl.BlockSpec((1, C), lambda b: (0, 0)),
        ],
        out_specs=pl.BlockSpec((1, H, W, C), lambda b: (b, 0, 0, 0)),
        out_shape=jax.ShapeDtypeStruct((B, H, W, C), jnp.bfloat16),
        scratch_shapes=[pltpu.VMEM((H + 2, W + 2, C), jnp.bfloat16)],
    )(x_nhwc, w1cat, b1, w2t, b2)


def _linears_body(fp_ref, wp_ref, wf_ref, b_ref, out_ref):
    pf = jax.lax.dot_general(
        fp_ref[...], wp_ref[...], (((1,), (1,)), ((), ())),
        preferred_element_type=jnp.float32)
    pf = pf.astype(jnp.bfloat16)
    out = jax.lax.dot_general(
        pf, wf_ref[...], (((1,), (1,)), ((), ())),
        preferred_element_type=jnp.float32)
    out_ref[...] = out + b_ref[...]


def _linears(fp16, wq16, wf16, fuse_b):
    return pl.pallas_call(
        _linears_body,
        out_shape=jax.ShapeDtypeStruct((N, 256), jnp.float32),
    )(fp16, wq16, wf16, fuse_b[None, :])


def kernel(cnn_feature, wh, ct_01, ct_ind, ct_img_idx, conv1_w, conv1_b,
           conv2_w, conv2_b, trans_poly_w, trans_fuse_w, trans_fuse_b):
    mask = ct_01.reshape(-1)
    ct_ind_f = jnp.where(mask, ct_ind.reshape(-1), 0)
    img_f = jnp.where(mask, ct_img_idx.reshape(-1), 0)
    ct_x = jnp.clip(ct_ind_f % W, 0, W - 1)
    ct_y = jnp.clip(ct_ind_f // W, 0, H - 1)
    ct = jnp.stack([ct_x.astype(jnp.float32), ct_y.astype(jnp.float32)], axis=1)

    cvec = jnp.arange(2 * NUM_POINT, dtype=jnp.int32) * (H * W)
    whidx = (img_f * (2 * NUM_POINT * H * W) + ct_y * W + ct_x)[:, None] + cvec[None, :]
    gathered = _wh_gather_sc(wh.reshape(-1), whidx.reshape(-1, 128))
    ct_offset = gathered.reshape(N, NUM_POINT, 2)
    init_polys = ct_offset * INIT_STRIDE + ct[:, None, :]

    x_nhwc = jnp.transpose(cnn_feature, (0, 2, 3, 1))
    w1cat = jnp.transpose(conv1_w, (2, 3, 1, 0)).reshape(576, 256).astype(jnp.bfloat16)
    w2t = conv2_w[:, :, 0, 0].T.astype(jnp.bfloat16)
    feat = _conv_refine(x_nhwc, w1cat, conv1_b[None, :], w2t, conv2_b[None, :])

    points = jnp.concatenate([ct[:, None, :], init_polys], axis=1)
    px = points[..., 0].reshape(-1)
    py = points[..., 1].reshape(-1)
    ib = jnp.repeat(img_f * (H * W), P1)
    featbits = lax.bitcast_convert_type(
        feat.reshape(B * H * W, C // 2, 2), jnp.int32)
    featpts = _grid_sample_sc(featbits, px, py, ib)

    fp16 = featpts.reshape(N, P1 * C)
    wq16 = (trans_poly_w.astype(jnp.bfloat16)
            .reshape(512, C, P1).transpose(0, 2, 1).reshape(512, P1 * C))
    wf16 = trans_fuse_w.astype(jnp.bfloat16)
    offsets = _linears(fp16, wq16, wf16, trans_fuse_b).reshape(N, NUM_POINT, 2)

    coarse_polys = offsets * COARSE_STRIDE + init_polys
    return init_polys * DOWN_SAMPLE, coarse_polys * DOWN_SAMPLE, ct


# speedup vs baseline: 1.1922x; 1.1460x over previous
import functools

import jax
import jax.numpy as jnp
from jax import lax
from jax.experimental import pallas as pl
from jax.experimental.pallas import tpu as pltpu
from jax.experimental.pallas import tpu_sc as plsc

NUM_POINT = 128
INIT_STRIDE = 10.0
COARSE_STRIDE = 4.0
DOWN_SAMPLE = 4.0

B, C, H, W = 4, 64, 128, 128
MAXOBJ = 128
N = B * MAXOBJ
P1 = NUM_POINT + 1
NPTS = N * P1
NWORK = 32
WH_PER_W = N * NUM_POINT * 2 // NWORK // 128
GS_PER_W = NPTS // NWORK
GS_CHUNKS = GS_PER_W // 16

def _wh_gather_body(wh_hbm, idx_hbm, out_hbm, idx_v, val_v, sem):
    wid = lax.axis_index("s") * 2 + lax.axis_index("c")
    base = wid * WH_PER_W
    pltpu.sync_copy(idx_hbm.at[pl.ds(base, WH_PER_W)], idx_v)
    descs = []
    for j in range(WH_PER_W):
        descs.append(pltpu.async_copy(wh_hbm.at[idx_v.at[j]], val_v.at[j], sem))
    for d in descs:
        d.wait()
    pltpu.sync_copy(val_v, out_hbm.at[pl.ds(base, WH_PER_W)])


def _wh_gather_sc(wh_flat, whidx):
    k = pl.kernel(
        _wh_gather_body,
        out_type=jax.ShapeDtypeStruct((N * 2 * NUM_POINT // 128, 128), jnp.float32),
        mesh=plsc.VectorSubcoreMesh(core_axis_name="c", subcore_axis_name="s"),
        scratch_types=[
            pltpu.VMEM((WH_PER_W, 128), jnp.int32),
            pltpu.VMEM((WH_PER_W, 128), jnp.float32),
            pltpu.SemaphoreType.DMA,
        ],
    )
    return k(wh_flat, whidx)


GS_CH = 48
GS_NCH = GS_PER_W // GS_CH
_ILV = plsc.PackFormat.INTERLEAVED


def _gs_stage(px_v, py_v, ib_v, idx_v, w_v, feat_hbm, tap_v, sems, base, ci, slot):
    for sub in range(GS_CH // 16):
        off = ci * GS_CH + sub * 16
        px = px_v[pl.ds(off, 16)]
        py = py_v[pl.ds(off, 16)]
        ib = ib_v[pl.ds(off, 16)]
        ix = px - 0.5
        iy = py - 0.5
        xt = ix.astype(jnp.int32)
        yt = iy.astype(jnp.int32)
        x0 = jnp.where(ix < xt.astype(jnp.float32), xt - 1, xt)
        y0 = jnp.where(iy < yt.astype(jnp.float32), yt - 1, yt)
        wx1 = ix - x0.astype(jnp.float32)
        wy1 = iy - y0.astype(jnp.float32)
        wx0 = 1.0 - wx1
        wy0 = 1.0 - wy1
        zero16 = jnp.zeros((16,), jnp.float32)
        for t, (dx, dy, wx, wy) in enumerate(
                ((0, 0, wx0, wy0), (1, 0, wx1, wy0),
                 (0, 1, wx0, wy1), (1, 1, wx1, wy1))):
            xi = x0 + dx
            yi = y0 + dy
            ok = (xi >= 0) & (xi < W) & (yi >= 0) & (yi < H)
            xc = jnp.minimum(jnp.maximum(xi, 0), W - 1)
            yc = jnp.minimum(jnp.maximum(yi, 0), H - 1)
            idx_v[slot, t, pl.ds(sub * 16, 16)] = ib + yc * W + xc
            w_v[slot, t, pl.ds(sub * 16, 16)] = jnp.where(ok, wx * wy, zero16)
    for t in range(4):
        pltpu.async_copy(feat_hbm.at[idx_v.at[slot, t]],
                         tap_v.at[slot, t], sems.at[slot])


def _gs_wait(feat_hbm, idx_v, tap_v, sems, slot):
    for t in range(4):
        pltpu.make_async_copy(feat_hbm.at[idx_v.at[slot, t]],
                              tap_v.at[slot, t], sems.at[slot]).wait()


def _gs_combine(w_v, tap_v, out_v, out_hbm, base, ci, slot):
    for sub in range(GS_CH // 16):
        wrows = [w_v[slot, t, pl.ds(sub * 16, 16)] for t in range(4)]
        for p in range(16):
            pt = sub * 16 + p
            a0 = jnp.zeros((32,), jnp.bfloat16)
            a1 = jnp.zeros((32,), jnp.bfloat16)
            for t in range(4):
                ws = jnp.broadcast_to(wrows[t][p], (16,))
                wpb = plsc.pack(ws, ws, format=_ILV)
                r0 = plsc.bitcast(tap_v[slot, t, pt, pl.ds(0, 16)], jnp.bfloat16)
                r1 = plsc.bitcast(tap_v[slot, t, pt, pl.ds(16, 16)], jnp.bfloat16)
                a0 = a0 + wpb * r0
                a1 = a1 + wpb * r1
            out_v[pt, pl.ds(0, 32)] = a0
            out_v[pt, pl.ds(32, 32)] = a1
    pltpu.sync_copy(out_v, out_hbm.at[pl.ds(base + ci * GS_CH, GS_CH)])


def _grid_sample_body(feat_hbm, px_hbm, py_hbm, ib_hbm, out_hbm,
                      px_v, py_v, ib_v, idx_v, w_v, tap_v, out_v, sems):
    wid = lax.axis_index("s") * 2 + lax.axis_index("c")
    base = wid * GS_PER_W
    pltpu.sync_copy(px_hbm.at[pl.ds(base, GS_PER_W)], px_v)
    pltpu.sync_copy(py_hbm.at[pl.ds(base, GS_PER_W)], py_v)
    pltpu.sync_copy(ib_hbm.at[pl.ds(base, GS_PER_W)], ib_v)

    _gs_stage(px_v, py_v, ib_v, idx_v, w_v, feat_hbm, tap_v, sems, base, 0, 0)

    def pair(j, carry):
        ci = j * 2
        _gs_stage(px_v, py_v, ib_v, idx_v, w_v, feat_hbm, tap_v, sems,
                  base, ci + 1, 1)
        _gs_wait(feat_hbm, idx_v, tap_v, sems, 0)
        _gs_combine(w_v, tap_v, out_v, out_hbm, base, ci, 0)
        _gs_stage(px_v, py_v, ib_v, idx_v, w_v, feat_hbm, tap_v, sems,
                  base, ci + 2, 0)
        _gs_wait(feat_hbm, idx_v, tap_v, sems, 1)
        _gs_combine(w_v, tap_v, out_v, out_hbm, base, ci + 1, 1)
        return carry

    lax.fori_loop(0, (GS_NCH - 1) // 2, pair, 0)
    _gs_wait(feat_hbm, idx_v, tap_v, sems, 0)
    _gs_combine(w_v, tap_v, out_v, out_hbm, base, GS_NCH - 1, 0)


def _grid_sample_sc(feat_rows, px, py, ib):
    k = pl.kernel(
        _grid_sample_body,
        out_type=jax.ShapeDtypeStruct((NPTS, C), jnp.bfloat16),
        mesh=plsc.VectorSubcoreMesh(core_axis_name="c", subcore_axis_name="s"),
        scratch_types=[
            pltpu.VMEM((GS_PER_W,), jnp.float32),
            pltpu.VMEM((GS_PER_W,), jnp.float32),
            pltpu.VMEM((GS_PER_W,), jnp.int32),
            pltpu.VMEM((2, 4, GS_CH), jnp.int32),
            pltpu.VMEM((2, 4, GS_CH), jnp.float32),
            pltpu.VMEM((2, 4, GS_CH, C // 2), jnp.int32),
            pltpu.VMEM((GS_CH, C), jnp.bfloat16),
            pltpu.SemaphoreType.DMA((2,)),
        ],
        compiler_params=pltpu.CompilerParams(needs_layout_passes=False,
                                             use_tc_tiling_on_sc=False),
    )
    return k(feat_rows, px, py, ib)


def _conv_body(x_ref, w1_ref, b1_ref, w2_ref, b2_ref, out_ref, pad_ref):
    pad_ref[...] = jnp.zeros_like(pad_ref)
    pad_ref[1:H + 1, 1:W + 1, :] = x_ref[0].astype(jnp.bfloat16)
    for rb in range(8):
        r0 = rb * 16
        taps = []
        for dy in range(3):
            for dx in range(3):
                taps.append(
                    pad_ref[r0 + dy:r0 + dy + 16, dx:dx + W, :].reshape(16 * W, C))
        a = jnp.concatenate(taps, axis=1)
        acc = jnp.dot(a, w1_ref[...], preferred_element_type=jnp.float32)
        acc = jnp.maximum(acc + b1_ref[...], 0.0).astype(jnp.bfloat16)
        o = jnp.dot(acc, w2_ref[...], preferred_element_type=jnp.float32)
        o = o + b2_ref[...]
        out_ref[0, r0:r0 + 16] = o.reshape(16, W, C).astype(jnp.bfloat16)


def _conv_refine(x_nhwc, w1cat, b1, w2t, b2):
    return pl.pallas_call(
        _conv_body,
        grid=(B,),
        in_specs=[
            pl.BlockSpec((1, H, W, C), lambda b: (b, 0, 0, 0)),
            pl.BlockSpec((576, 256), lambda b: (0, 0)),
            pl.BlockSpec((1, 256), lambda b: (0, 0)),
            pl.BlockSpec((256, C), lambda b: (0, 0)),
            pl.BlockSpec((1, C), lambda b: (0, 0)),
        ],
        out_specs=pl.BlockSpec((1, H, W, C), lambda b: (b, 0, 0, 0)),
        out_shape=jax.ShapeDtypeStruct((B, H, W, C), jnp.bfloat16),
        scratch_shapes=[pltpu.VMEM((H + 2, W + 2, C), jnp.bfloat16)],
    )(x_nhwc, w1cat, b1, w2t, b2)


def _linears_body(fp_ref, wp_ref, wf_ref, b_ref, out_ref):
    pf = jax.lax.dot_general(
        fp_ref[...], wp_ref[...], (((1,), (1,)), ((), ())),
        preferred_element_type=jnp.float32)
    pf = pf.astype(jnp.bfloat16)
    out = jax.lax.dot_general(
        pf, wf_ref[...], (((1,), (1,)), ((), ())),
        preferred_element_type=jnp.float32)
    out_ref[...] = out + b_ref[...]


def _linears(fp16, wq16, wf16, fuse_b):
    return pl.pallas_call(
        _linears_body,
        out_shape=jax.ShapeDtypeStruct((N, 256), jnp.float32),
    )(fp16, wq16, wf16, fuse_b[None, :])


def kernel(cnn_feature, wh, ct_01, ct_ind, ct_img_idx, conv1_w, conv1_b,
           conv2_w, conv2_b, trans_poly_w, trans_fuse_w, trans_fuse_b):
    mask = ct_01.reshape(-1)
    ct_ind_f = jnp.where(mask, ct_ind.reshape(-1), 0)
    img_f = jnp.where(mask, ct_img_idx.reshape(-1), 0)
    ct_x = jnp.clip(ct_ind_f % W, 0, W - 1)
    ct_y = jnp.clip(ct_ind_f // W, 0, H - 1)
    ct = jnp.stack([ct_x.astype(jnp.float32), ct_y.astype(jnp.float32)], axis=1)

    cvec = jnp.arange(2 * NUM_POINT, dtype=jnp.int32) * (H * W)
    whidx = (img_f * (2 * NUM_POINT * H * W) + ct_y * W + ct_x)[:, None] + cvec[None, :]
    gathered = _wh_gather_sc(wh.reshape(-1), whidx.reshape(-1, 128))
    ct_offset = gathered.reshape(N, NUM_POINT, 2)
    init_polys = ct_offset * INIT_STRIDE + ct[:, None, :]

    x_nhwc = jnp.transpose(cnn_feature, (0, 2, 3, 1))
    w1cat = jnp.transpose(conv1_w, (2, 3, 1, 0)).reshape(576, 256).astype(jnp.bfloat16)
    w2t = conv2_w[:, :, 0, 0].T.astype(jnp.bfloat16)
    feat = _conv_refine(x_nhwc, w1cat, conv1_b[None, :], w2t, conv2_b[None, :])

    points = jnp.concatenate([ct[:, None, :], init_polys], axis=1)
    px = points[..., 0].reshape(-1)
    py = points[..., 1].reshape(-1)
    ib = jnp.repeat(img_f * (H * W), P1)
    featbits = lax.bitcast_convert_type(
        feat.reshape(B * H * W, C // 2, 2), jnp.int32)
    featpts = _grid_sample_sc(featbits, px, py, ib)

    fp16 = featpts.reshape(N, P1 * C)
    wq16 = (trans_poly_w.astype(jnp.bfloat16)
            .reshape(512, C, P1).transpose(0, 2, 1).reshape(512, P1 * C))
    wf16 = trans_fuse_w.astype(jnp.bfloat16)
    offsets = _linears(fp16, wq16, wf16, trans_fuse_b).reshape(N, NUM_POINT, 2)

    coarse_polys = offsets * COARSE_STRIDE + init_polys
    return init_polys * DOWN_SAMPLE, coarse_polys * DOWN_SAMPLE, ct


# speedup vs baseline: 1.3984x; 1.1730x over previous
import functools

import jax
import jax.numpy as jnp
from jax import lax
from jax.experimental import pallas as pl
from jax.experimental.pallas import tpu as pltpu
from jax.experimental.pallas import tpu_sc as plsc

NUM_POINT = 128
INIT_STRIDE = 10.0
COARSE_STRIDE = 4.0
DOWN_SAMPLE = 4.0

B, C, H, W = 4, 64, 128, 128
MAXOBJ = 128
N = B * MAXOBJ
P1 = NUM_POINT + 1
NPTS = N * P1
NWORK = 32
WH_PER_W = N * NUM_POINT * 2 // NWORK // 128
GS_PER_W = NPTS // NWORK
GS_CHUNKS = GS_PER_W // 16

def _wh_gather_body(wh_hbm, idx_hbm, out_hbm, idx_v, val_v, sem):
    wid = lax.axis_index("s") * 2 + lax.axis_index("c")
    base = wid * WH_PER_W
    pltpu.sync_copy(idx_hbm.at[pl.ds(base, WH_PER_W)], idx_v)
    descs = []
    for j in range(WH_PER_W):
        descs.append(pltpu.async_copy(wh_hbm.at[idx_v.at[j]], val_v.at[j], sem))
    for d in descs:
        d.wait()
    pltpu.sync_copy(val_v, out_hbm.at[pl.ds(base, WH_PER_W)])


def _wh_gather_sc(wh_flat, whidx):
    k = pl.kernel(
        _wh_gather_body,
        out_type=jax.ShapeDtypeStruct((N * 2 * NUM_POINT // 128, 128), jnp.float32),
        mesh=plsc.VectorSubcoreMesh(core_axis_name="c", subcore_axis_name="s"),
        scratch_types=[
            pltpu.VMEM((WH_PER_W, 128), jnp.int32),
            pltpu.VMEM((WH_PER_W, 128), jnp.float32),
            pltpu.SemaphoreType.DMA,
        ],
    )
    return k(wh_flat, whidx)


GS_CH = 48
GS_NCH = GS_PER_W // GS_CH
_ILV = plsc.PackFormat.INTERLEAVED


def _gs_stage(px_v, py_v, ib_v, idx_v, w_v, feat_hbm, tap_v, sems, base, ci, slot):
    for sub in range(GS_CH // 16):
        off = ci * GS_CH + sub * 16
        px = px_v[pl.ds(off, 16)]
        py = py_v[pl.ds(off, 16)]
        ib = ib_v[pl.ds(off, 16)]
        ix = px - 0.5
        iy = py - 0.5
        xt = ix.astype(jnp.int32)
        yt = iy.astype(jnp.int32)
        x0 = jnp.where(ix < xt.astype(jnp.float32), xt - 1, xt)
        y0 = jnp.where(iy < yt.astype(jnp.float32), yt - 1, yt)
        wx1 = ix - x0.astype(jnp.float32)
        wy1 = iy - y0.astype(jnp.float32)
        wx0 = 1.0 - wx1
        wy0 = 1.0 - wy1
        zero16 = jnp.zeros((16,), jnp.float32)
        for t, (dx, dy, wx, wy) in enumerate(
                ((0, 0, wx0, wy0), (1, 0, wx1, wy0),
                 (0, 1, wx0, wy1), (1, 1, wx1, wy1))):
            xi = x0 + dx
            yi = y0 + dy
            ok = (xi >= 0) & (xi < W) & (yi >= 0) & (yi < H)
            xc = jnp.minimum(jnp.maximum(xi, 0), W - 1)
            yc = jnp.minimum(jnp.maximum(yi, 0), H - 1)
            idx_v[slot, t, pl.ds(sub * 16, 16)] = ib + yc * W + xc
            w_v[slot, t, pl.ds(sub * 16, 16)] = jnp.where(ok, wx * wy, zero16)
    for t in range(4):
        pltpu.async_copy(feat_hbm.at[idx_v.at[slot, t]],
                         tap_v.at[slot, t], sems.at[slot])


def _gs_wait(feat_hbm, idx_v, tap_v, sems, slot):
    for t in range(4):
        pltpu.make_async_copy(feat_hbm.at[idx_v.at[slot, t]],
                              tap_v.at[slot, t], sems.at[slot]).wait()


def _gs_combine(w_v, tap_v, out_v, out_hbm, base, ci, slot):
    for sub in range(GS_CH // 16):
        wrows = [w_v[slot, t, pl.ds(sub * 16, 16)] for t in range(4)]
        for p in range(16):
            pt = sub * 16 + p
            a0 = jnp.zeros((32,), jnp.bfloat16)
            a1 = jnp.zeros((32,), jnp.bfloat16)
            for t in range(4):
                ws = jnp.broadcast_to(wrows[t][p], (16,))
                wpb = plsc.pack(ws, ws, format=_ILV)
                r0 = plsc.bitcast(tap_v[slot, t, pt, pl.ds(0, 16)], jnp.bfloat16)
                r1 = plsc.bitcast(tap_v[slot, t, pt, pl.ds(16, 16)], jnp.bfloat16)
                a0 = a0 + wpb * r0
                a1 = a1 + wpb * r1
            out_v[pt, pl.ds(0, 32)] = a0
            out_v[pt, pl.ds(32, 32)] = a1
    pltpu.sync_copy(out_v, out_hbm.at[pl.ds(base + ci * GS_CH, GS_CH)])


def _grid_sample_body(feat_hbm, px_hbm, py_hbm, ib_hbm, out_hbm,
                      px_v, py_v, ib_v, idx_v, w_v, tap_v, out_v, sems):
    wid = lax.axis_index("s") * 2 + lax.axis_index("c")
    base = wid * GS_PER_W
    pltpu.sync_copy(px_hbm.at[pl.ds(base, GS_PER_W)], px_v)
    pltpu.sync_copy(py_hbm.at[pl.ds(base, GS_PER_W)], py_v)
    pltpu.sync_copy(ib_hbm.at[pl.ds(base, GS_PER_W)], ib_v)

    _gs_stage(px_v, py_v, ib_v, idx_v, w_v, feat_hbm, tap_v, sems, base, 0, 0)

    def pair(j, carry):
        ci = j * 2
        _gs_stage(px_v, py_v, ib_v, idx_v, w_v, feat_hbm, tap_v, sems,
                  base, ci + 1, 1)
        _gs_wait(feat_hbm, idx_v, tap_v, sems, 0)
        _gs_combine(w_v, tap_v, out_v, out_hbm, base, ci, 0)
        _gs_stage(px_v, py_v, ib_v, idx_v, w_v, feat_hbm, tap_v, sems,
                  base, ci + 2, 0)
        _gs_wait(feat_hbm, idx_v, tap_v, sems, 1)
        _gs_combine(w_v, tap_v, out_v, out_hbm, base, ci + 1, 1)
        return carry

    lax.fori_loop(0, (GS_NCH - 1) // 2, pair, 0)
    _gs_wait(feat_hbm, idx_v, tap_v, sems, 0)
    _gs_combine(w_v, tap_v, out_v, out_hbm, base, GS_NCH - 1, 0)


def _grid_sample_sc(feat_rows, px, py, ib):
    k = pl.kernel(
        _grid_sample_body,
        out_type=jax.ShapeDtypeStruct((NPTS, C), jnp.bfloat16),
        mesh=plsc.VectorSubcoreMesh(core_axis_name="c", subcore_axis_name="s"),
        scratch_types=[
            pltpu.VMEM((GS_PER_W,), jnp.float32),
            pltpu.VMEM((GS_PER_W,), jnp.float32),
            pltpu.VMEM((GS_PER_W,), jnp.int32),
            pltpu.VMEM((2, 4, GS_CH), jnp.int32),
            pltpu.VMEM((2, 4, GS_CH), jnp.float32),
            pltpu.VMEM((2, 4, GS_CH, C // 2), jnp.int32),
            pltpu.VMEM((GS_CH, C), jnp.bfloat16),
            pltpu.SemaphoreType.DMA((2,)),
        ],
        compiler_params=pltpu.CompilerParams(needs_layout_passes=False,
                                             use_tc_tiling_on_sc=False),
    )
    return k(feat_rows, px, py, ib)


def _conv_body(x_ref, w1_ref, b1_ref, w2_ref, b2_ref, out_ref, pad_ref):
    pad_ref[...] = jnp.zeros_like(pad_ref)
    pad_ref[1:H + 1, 1:W + 1, :] = x_ref[0].astype(jnp.bfloat16)
    for rb in range(8):
        r0 = rb * 16
        taps = []
        for dy in range(3):
            for dx in range(3):
                taps.append(
                    pad_ref[r0 + dy:r0 + dy + 16, dx:dx + W, :].reshape(16 * W, C))
        a = jnp.concatenate(taps, axis=1)
        acc = jnp.dot(a, w1_ref[...], preferred_element_type=jnp.float32)
        acc = jnp.maximum(acc + b1_ref[...], 0.0).astype(jnp.bfloat16)
        o = jnp.dot(acc, w2_ref[...], preferred_element_type=jnp.float32)
        o = o + b2_ref[...]
        out_ref[0, r0:r0 + 16] = o.reshape(16, W, C).astype(jnp.bfloat16)


def _conv_refine(x_nhwc, w1cat, b1, w2t, b2):
    return pl.pallas_call(
        _conv_body,
        grid=(B,),
        in_specs=[
            pl.BlockSpec((1, H, W, C), lambda b: (b, 0, 0, 0)),
            pl.BlockSpec((576, 256), lambda b: (0, 0)),
            pl.BlockSpec((1, 256), lambda b: (0, 0)),
            pl.BlockSpec((256, C), lambda b: (0, 0)),
            pl.BlockSpec((1, C), lambda b: (0, 0)),
        ],
        out_specs=pl.BlockSpec((1, H, W, C), lambda b: (b, 0, 0, 0)),
        out_shape=jax.ShapeDtypeStruct((B, H, W, C), jnp.bfloat16),
        scratch_shapes=[pltpu.VMEM((H + 2, W + 2, C), jnp.bfloat16)],
    )(x_nhwc, w1cat, b1, w2t, b2)


def _linears_body(fp_ref, wp_ref, wf_ref, b_ref, ip4_ref, out_ref):
    pf = jax.lax.dot_general(
        fp_ref[...], wp_ref[...], (((1,), (1,)), ((), ())),
        preferred_element_type=jnp.float32)
    pf = pf.astype(jnp.bfloat16)
    out = jax.lax.dot_general(
        pf, wf_ref[...], (((1,), (1,)), ((), ())),
        preferred_element_type=jnp.float32)
    out_ref[...] = ((out + b_ref[...]) * (COARSE_STRIDE * DOWN_SAMPLE)
                    + ip4_ref[...])


def _linears(fp16, wq16, wf16, fuse_b, ip4):
    return pl.pallas_call(
        _linears_body,
        out_shape=jax.ShapeDtypeStruct((N, 256), jnp.float32),
    )(fp16, wq16, wf16, fuse_b[None, :], ip4)


def kernel(cnn_feature, wh, ct_01, ct_ind, ct_img_idx, conv1_w, conv1_b,
           conv2_w, conv2_b, trans_poly_w, trans_fuse_w, trans_fuse_b):
    mask = ct_01.reshape(-1)
    ct_ind_f = jnp.where(mask, ct_ind.reshape(-1), 0)
    img_f = jnp.where(mask, ct_img_idx.reshape(-1), 0)
    ct_x = jnp.clip(ct_ind_f % W, 0, W - 1)
    ct_y = jnp.clip(ct_ind_f // W, 0, H - 1)
    ctxf = ct_x.astype(jnp.float32)
    ctyf = ct_y.astype(jnp.float32)
    ct = jnp.stack([ctxf, ctyf], axis=1)

    cvec = jnp.arange(2 * NUM_POINT, dtype=jnp.int32) * (H * W)
    whidx = (img_f * (2 * NUM_POINT * H * W) + ct_y * W + ct_x)[:, None] + cvec[None, :]
    gath1024 = _wh_gather_sc(wh.reshape(-1), whidx.reshape(-1, 128))
    ct_tile = jnp.repeat(jnp.tile(jnp.stack([ctxf, ctyf], -1), (1, 64)), 2, axis=0)
    ipolys = gath1024 * INIT_STRIDE + ct_tile
    ip4 = (ipolys * DOWN_SAMPLE).reshape(N, 2 * NUM_POINT)

    x_nhwc = jnp.transpose(cnn_feature, (0, 2, 3, 1))
    w1cat = jnp.transpose(conv1_w, (2, 3, 1, 0)).reshape(576, 256).astype(jnp.bfloat16)
    w2t = conv2_w[:, :, 0, 0].T.astype(jnp.bfloat16)
    feat = _conv_refine(x_nhwc, w1cat, conv1_b[None, :], w2t, conv2_b[None, :])
    featbits = lax.bitcast_convert_type(
        feat.reshape(B * H * W, C // 2, 2), jnp.int32)

    xy3 = ipolys.reshape(N, NUM_POINT, 2)
    px = jnp.concatenate([ctxf[:, None], xy3[..., 0]], axis=1).reshape(-1)
    py = jnp.concatenate([ctyf[:, None], xy3[..., 1]], axis=1).reshape(-1)
    ib = jnp.repeat(img_f * (H * W), P1)
    featpts = _grid_sample_sc(featbits, px, py, ib)

    fp16 = featpts.reshape(N, P1 * C)
    wq16 = (trans_poly_w.astype(jnp.bfloat16)
            .reshape(512, C, P1).transpose(0, 2, 1).reshape(512, P1 * C))
    wf16 = trans_fuse_w.astype(jnp.bfloat16)
    coarse4 = _linears(fp16, wq16, wf16, trans_fuse_b, ip4)

    return (ip4.reshape(N, NUM_POINT, 2),
            coarse4.reshape(N, NUM_POINT, 2), ct)


# speedup vs baseline: 1.4442x; 1.0328x over previous
import functools

import jax
import jax.numpy as jnp
from jax import lax
from jax.experimental import pallas as pl
from jax.experimental.pallas import tpu as pltpu
from jax.experimental.pallas import tpu_sc as plsc

NUM_POINT = 128
INIT_STRIDE = 10.0
COARSE_STRIDE = 4.0
DOWN_SAMPLE = 4.0

B, C, H, W = 4, 64, 128, 128
MAXOBJ = 128
N = B * MAXOBJ
P1 = NUM_POINT + 1
NPTS = N * P1
NWORK = 32
WH_PER_W = N * NUM_POINT * 2 // NWORK // 128
GS_PER_W = NPTS // NWORK
GS_CHUNKS = GS_PER_W // 16

def _wh_gather_body(wh_hbm, idx_hbm, out_hbm, idx_v, val_v, sem):
    wid = lax.axis_index("s") * 2 + lax.axis_index("c")
    base = wid * WH_PER_W
    pltpu.sync_copy(idx_hbm.at[pl.ds(base, WH_PER_W)], idx_v)
    descs = []
    for j in range(WH_PER_W):
        descs.append(pltpu.async_copy(wh_hbm.at[idx_v.at[j]], val_v.at[j], sem))
    for d in descs:
        d.wait()
    pltpu.sync_copy(val_v, out_hbm.at[pl.ds(base, WH_PER_W)])


def _wh_gather_sc(wh_flat, whidx):
    k = pl.kernel(
        _wh_gather_body,
        out_type=jax.ShapeDtypeStruct((N * 2 * NUM_POINT // 128, 128), jnp.float32),
        mesh=plsc.VectorSubcoreMesh(core_axis_name="c", subcore_axis_name="s"),
        scratch_types=[
            pltpu.VMEM((WH_PER_W, 128), jnp.int32),
            pltpu.VMEM((WH_PER_W, 128), jnp.float32),
            pltpu.SemaphoreType.DMA,
        ],
    )
    return k(wh_flat, whidx)


GS_CH = 48
GS_NCH = GS_PER_W // GS_CH
_ILV = plsc.PackFormat.INTERLEAVED


def _gs_stage(px_v, py_v, ib_v, idx_v, w_v, p_v, feat_hbm, tap_v, sems,
              base, ci, slot):
    for sub in range(GS_CH // 16):
        off = ci * GS_CH + sub * 16
        px = px_v[pl.ds(off, 16)]
        py = py_v[pl.ds(off, 16)]
        ib = ib_v[pl.ds(off, 16)]
        ix = px - 0.5
        iy = py - 0.5
        xt = ix.astype(jnp.int32)
        yt = iy.astype(jnp.int32)
        x0 = jnp.where(ix < xt.astype(jnp.float32), xt - 1, xt)
        y0 = jnp.where(iy < yt.astype(jnp.float32), yt - 1, yt)
        wx1 = ix - x0.astype(jnp.float32)
        wy1 = iy - y0.astype(jnp.float32)
        wx0 = 1.0 - wx1
        wy0 = 1.0 - wy1
        zero16 = jnp.zeros((16,), jnp.float32)
        for t, (dx, dy, wx, wy) in enumerate(
                ((0, 0, wx0, wy0), (1, 0, wx1, wy0),
                 (0, 1, wx0, wy1), (1, 1, wx1, wy1))):
            xi = x0 + dx
            yi = y0 + dy
            ok = (xi >= 0) & (xi < W) & (yi >= 0) & (yi < H)
            xc = jnp.minimum(jnp.maximum(xi, 0), W - 1)
            yc = jnp.minimum(jnp.maximum(yi, 0), H - 1)
            q = ib + yc * W + xc
            idx_v[slot, t, pl.ds(sub * 16, 16)] = q >> 1
            p_v[slot, t, pl.ds(sub * 16, 16)] = (q & 1) * 64
            w_v[slot, t, pl.ds(sub * 16, 16)] = jnp.where(ok, wx * wy, zero16)
    for t in range(4):
        pltpu.async_copy(feat_hbm.at[idx_v.at[slot, t]],
                         tap_v.at[slot, t], sems.at[slot])


def _gs_wait(feat_hbm, idx_v, tap_v, sems, slot):
    for t in range(4):
        pltpu.make_async_copy(feat_hbm.at[idx_v.at[slot, t]],
                              tap_v.at[slot, t], sems.at[slot]).wait()


def _gs_combine(w_v, p_v, tap_v, out_v, out_hbm, base, ci, slot):
    for sub in range(GS_CH // 16):
        wrows = [w_v[slot, t, pl.ds(sub * 16, 16)] for t in range(4)]
        prows = [p_v[slot, t, pl.ds(sub * 16, 16)] for t in range(4)]
        for p in range(16):
            pt = sub * 16 + p
            acc = [jnp.zeros((16,), jnp.float32) for _ in range(4)]
            for t in range(4):
                ws = jnp.broadcast_to(wrows[t][p], (16,))
                par = prows[t][p]
                for g in range(4):
                    v = tap_v[slot, t, pt, pl.ds(par + 16 * g, 16)]
                    acc[g] = acc[g] + ws * v
            out_v[pt, pl.ds(0, 32)] = plsc.pack(acc[0], acc[1], format=_ILV)
            out_v[pt, pl.ds(32, 32)] = plsc.pack(acc[2], acc[3], format=_ILV)
    pltpu.sync_copy(out_v, out_hbm.at[pl.ds(base + ci * GS_CH, GS_CH)])


def _grid_sample_body(feat_hbm, px_hbm, py_hbm, ib_hbm, out_hbm,
                      px_v, py_v, ib_v, idx_v, w_v, p_v, tap_v, out_v, sems):
    wid = lax.axis_index("s") * 2 + lax.axis_index("c")
    base = wid * GS_PER_W
    pltpu.sync_copy(px_hbm.at[pl.ds(base, GS_PER_W)], px_v)
    pltpu.sync_copy(py_hbm.at[pl.ds(base, GS_PER_W)], py_v)
    pltpu.sync_copy(ib_hbm.at[pl.ds(base, GS_PER_W)], ib_v)

    _gs_stage(px_v, py_v, ib_v, idx_v, w_v, p_v, feat_hbm, tap_v, sems,
              base, 0, 0)

    def pair(j, carry):
        ci = j * 2
        _gs_stage(px_v, py_v, ib_v, idx_v, w_v, p_v, feat_hbm, tap_v, sems,
                  base, ci + 1, 1)
        _gs_wait(feat_hbm, idx_v, tap_v, sems, 0)
        _gs_combine(w_v, p_v, tap_v, out_v, out_hbm, base, ci, 0)
        _gs_stage(px_v, py_v, ib_v, idx_v, w_v, p_v, feat_hbm, tap_v, sems,
                  base, ci + 2, 0)
        _gs_wait(feat_hbm, idx_v, tap_v, sems, 1)
        _gs_combine(w_v, p_v, tap_v, out_v, out_hbm, base, ci + 1, 1)
        return carry

    lax.fori_loop(0, (GS_NCH - 1) // 2, pair, 0)
    _gs_wait(feat_hbm, idx_v, tap_v, sems, 0)
    _gs_combine(w_v, p_v, tap_v, out_v, out_hbm, base, GS_NCH - 1, 0)


def _grid_sample_sc(feat_rows, px, py, ib):
    k = pl.kernel(
        _grid_sample_body,
        out_type=jax.ShapeDtypeStruct((NPTS, C), jnp.bfloat16),
        mesh=plsc.VectorSubcoreMesh(core_axis_name="c", subcore_axis_name="s"),
        scratch_types=[
            pltpu.VMEM((GS_PER_W,), jnp.float32),
            pltpu.VMEM((GS_PER_W,), jnp.float32),
            pltpu.VMEM((GS_PER_W,), jnp.int32),
            pltpu.VMEM((2, 4, GS_CH), jnp.int32),
            pltpu.VMEM((2, 4, GS_CH), jnp.float32),
            pltpu.VMEM((2, 4, GS_CH), jnp.int32),
            pltpu.VMEM((2, 4, GS_CH, 2 * C), jnp.float32),
            pltpu.VMEM((GS_CH, C), jnp.bfloat16),
            pltpu.SemaphoreType.DMA((2,)),
        ],
        compiler_params=pltpu.CompilerParams(needs_layout_passes=False,
                                             use_tc_tiling_on_sc=False),
    )
    return k(feat_rows, px, py, ib)


def _conv_body(x_ref, w1_ref, b1_ref, w2_ref, b2_ref, out_ref, pad_ref):
    pad_ref[...] = jnp.zeros_like(pad_ref)
    pad_ref[1:H + 1, 1:W + 1, :] = x_ref[0].astype(jnp.bfloat16)
    for rb in range(8):
        r0 = rb * 16
        taps = []
        for dy in range(3):
            for dx in range(3):
                taps.append(
                    pad_ref[r0 + dy:r0 + dy + 16, dx:dx + W, :].reshape(16 * W, C))
        a = jnp.concatenate(taps, axis=1)
        acc = jnp.dot(a, w1_ref[...], preferred_element_type=jnp.float32)
        acc = jnp.maximum(acc + b1_ref[...], 0.0).astype(jnp.bfloat16)
        o = jnp.dot(acc, w2_ref[...], preferred_element_type=jnp.float32)
        o = o + b2_ref[...]
        o3 = o.reshape(1024, 2, C)
        out_ref[0, rb * 1024:rb * 1024 + 1024, 0:C] = o3[:, 0, :]
        out_ref[0, rb * 1024:rb * 1024 + 1024, C:2 * C] = o3[:, 1, :]


def _conv_refine(x_nhwc, w1cat, b1, w2t, b2):
    return pl.pallas_call(
        _conv_body,
        grid=(B,),
        in_specs=[
            pl.BlockSpec((1, H, W, C), lambda b: (b, 0, 0, 0)),
            pl.BlockSpec((576, 256), lambda b: (0, 0)),
            pl.BlockSpec((1, 256), lambda b: (0, 0)),
            pl.BlockSpec((256, C), lambda b: (0, 0)),
            pl.BlockSpec((1, C), lambda b: (0, 0)),
        ],
        out_specs=pl.BlockSpec((1, H * W * C // 128, 128), lambda b: (b, 0, 0)),
        out_shape=jax.ShapeDtypeStruct((B, H * W * C // 128, 128), jnp.float32),
        scratch_shapes=[pltpu.VMEM((H + 2, W + 2, C), jnp.bfloat16)],
    )(x_nhwc, w1cat, b1, w2t, b2)


def _linears_body(fp_ref, wp_ref, wf_ref, b_ref, ip4_ref, out_ref):
    pf = jax.lax.dot_general(
        fp_ref[...], wp_ref[...], (((1,), (1,)), ((), ())),
        preferred_element_type=jnp.float32)
    pf = pf.astype(jnp.bfloat16)
    out = jax.lax.dot_general(
        pf, wf_ref[...], (((1,), (1,)), ((), ())),
        preferred_element_type=jnp.float32)
    out_ref[...] = ((out + b_ref[...]) * (COARSE_STRIDE * DOWN_SAMPLE)
                    + ip4_ref[...])


def _linears(fp16, wq16, wf16, fuse_b, ip4):
    return pl.pallas_call(
        _linears_body,
        out_shape=jax.ShapeDtypeStruct((N, 256), jnp.float32),
    )(fp16, wq16, wf16, fuse_b[None, :], ip4)


def kernel(cnn_feature, wh, ct_01, ct_ind, ct_img_idx, conv1_w, conv1_b,
           conv2_w, conv2_b, trans_poly_w, trans_fuse_w, trans_fuse_b):
    mask = ct_01.reshape(-1)
    ct_ind_f = jnp.where(mask, ct_ind.reshape(-1), 0)
    img_f = jnp.where(mask, ct_img_idx.reshape(-1), 0)
    ct_x = jnp.clip(ct_ind_f % W, 0, W - 1)
    ct_y = jnp.clip(ct_ind_f // W, 0, H - 1)
    ctxf = ct_x.astype(jnp.float32)
    ctyf = ct_y.astype(jnp.float32)
    ct = jnp.stack([ctxf, ctyf], axis=1)

    cvec = jnp.arange(2 * NUM_POINT, dtype=jnp.int32) * (H * W)
    whidx = (img_f * (2 * NUM_POINT * H * W) + ct_y * W + ct_x)[:, None] + cvec[None, :]
    gath1024 = _wh_gather_sc(wh.reshape(-1), whidx.reshape(-1, 128))
    ct_tile = jnp.repeat(jnp.tile(jnp.stack([ctxf, ctyf], -1), (1, 64)), 2, axis=0)
    ipolys = gath1024 * INIT_STRIDE + ct_tile
    ip4 = (ipolys * DOWN_SAMPLE).reshape(N, 2 * NUM_POINT)

    x_nhwc = jnp.transpose(cnn_feature, (0, 2, 3, 1))
    w1cat = jnp.transpose(conv1_w, (2, 3, 1, 0)).reshape(576, 256).astype(jnp.bfloat16)
    w2t = conv2_w[:, :, 0, 0].T.astype(jnp.bfloat16)
    featrows = _conv_refine(x_nhwc, w1cat, conv1_b[None, :], w2t,
                            conv2_b[None, :]).reshape(B * H * W // 2, 2 * C)

    xy3 = ipolys.reshape(N, NUM_POINT, 2)
    px = jnp.concatenate([ctxf[:, None], xy3[..., 0]], axis=1).reshape(-1)
    py = jnp.concatenate([ctyf[:, None], xy3[..., 1]], axis=1).reshape(-1)
    ib = jnp.repeat(img_f * (H * W), P1)
    featpts = _grid_sample_sc(featrows, px, py, ib)

    fp16 = featpts.reshape(N, P1 * C)
    cord = (jnp.arange(C) // 32) * 32 + jnp.where(
        jnp.arange(C) % 2 == 0, (jnp.arange(C) % 32) // 2,
        16 + (jnp.arange(C) % 32) // 2)
    wq16 = (trans_poly_w.astype(jnp.bfloat16)
            .reshape(512, C, P1).transpose(0, 2, 1)[:, :, cord]
            .reshape(512, P1 * C))
    wf16 = trans_fuse_w.astype(jnp.bfloat16)
    coarse4 = _linears(fp16, wq16, wf16, trans_fuse_b, ip4)

    return (ip4.reshape(N, NUM_POINT, 2),
            coarse4.reshape(N, NUM_POINT, 2), ct)


# speedup vs baseline: 1.6654x; 1.1531x over previous
import functools

import jax
import jax.numpy as jnp
from jax import lax
from jax.experimental import pallas as pl
from jax.experimental.pallas import tpu as pltpu
from jax.experimental.pallas import tpu_sc as plsc

NUM_POINT = 128
INIT_STRIDE = 10.0
COARSE_STRIDE = 4.0
DOWN_SAMPLE = 4.0

B, C, H, W = 4, 64, 128, 128
MAXOBJ = 128
N = B * MAXOBJ
P1 = NUM_POINT + 1
NPTS = N * P1
NWORK = 32
WH_PER_W = N * NUM_POINT * 2 // NWORK // 128
GS_PER_W = NPTS // NWORK
GS_CHUNKS = GS_PER_W // 16

def _wh_gather_body(wh_hbm, idx_hbm, out_hbm, idx_v, val_v, sem):
    wid = lax.axis_index("s") * 2 + lax.axis_index("c")
    base = wid * WH_PER_W
    pltpu.sync_copy(idx_hbm.at[pl.ds(base, WH_PER_W)], idx_v)
    descs = []
    for j in range(WH_PER_W):
        descs.append(pltpu.async_copy(wh_hbm.at[idx_v.at[j]], val_v.at[j], sem))
    for d in descs:
        d.wait()
    pltpu.sync_copy(val_v, out_hbm.at[pl.ds(base, WH_PER_W)])


def _wh_gather_sc(wh_flat, whidx):
    k = pl.kernel(
        _wh_gather_body,
        out_type=jax.ShapeDtypeStruct((N * 2 * NUM_POINT // 128, 128), jnp.float32),
        mesh=plsc.VectorSubcoreMesh(core_axis_name="c", subcore_axis_name="s"),
        scratch_types=[
            pltpu.VMEM((WH_PER_W, 128), jnp.int32),
            pltpu.VMEM((WH_PER_W, 128), jnp.float32),
            pltpu.SemaphoreType.DMA,
        ],
    )
    return k(wh_flat, whidx)


GS_CH = 48
GS_NCH = GS_PER_W // GS_CH
_ILV = plsc.PackFormat.INTERLEAVED


def _gs_stage(gath_v, ctx_v, cty_v, ibr_v, base, n0, idx_v, w_v, p_v,
              feat_hbm, tap_v, sems, ci, slot):
    for sub in range(GS_CH // 16):
        off = ci * GS_CH + sub * 16
        pt = base + off + lax.iota(jnp.int32, 16)
        n16 = pt // P1
        r16 = pt - n16 * P1
        nl = n16 - n0
        ctx = plsc.load_gather(ctx_v, [nl])
        cty = plsc.load_gather(cty_v, [nl])
        ib = plsc.load_gather(ibr_v, [nl])
        cfx = (r16 - 1) * 2
        cfx = jnp.maximum(cfx, 0)
        rowx = 2 * nl + (cfx >> 7)
        lanex = cfx & 127
        offx = plsc.load_gather(gath_v, [rowx, lanex])
        cfy = cfx + 1
        rowy = 2 * nl + (cfy >> 7)
        laney = cfy & 127
        offy = plsc.load_gather(gath_v, [rowy, laney])
        is_ct = r16 == 0
        px = jnp.where(is_ct, ctx, offx * INIT_STRIDE + ctx)
        py = jnp.where(is_ct, cty, offy * INIT_STRIDE + cty)
        ix = px - 0.5
        iy = py - 0.5
        xt = ix.astype(jnp.int32)
        yt = iy.astype(jnp.int32)
        x0 = jnp.where(ix < xt.astype(jnp.float32), xt - 1, xt)
        y0 = jnp.where(iy < yt.astype(jnp.float32), yt - 1, yt)
        wx1 = ix - x0.astype(jnp.float32)
        wy1 = iy - y0.astype(jnp.float32)
        wx0 = 1.0 - wx1
        wy0 = 1.0 - wy1
        zero16 = jnp.zeros((16,), jnp.float32)
        for t, (dx, dy, wx, wy) in enumerate(
                ((0, 0, wx0, wy0), (1, 0, wx1, wy0),
                 (0, 1, wx0, wy1), (1, 1, wx1, wy1))):
            xi = x0 + dx
            yi = y0 + dy
            ok = (xi >= 0) & (xi < W) & (yi >= 0) & (yi < H)
            xc = jnp.minimum(jnp.maximum(xi, 0), W - 1)
            yc = jnp.minimum(jnp.maximum(yi, 0), H - 1)
            q = ib + yc * W + xc
            idx_v[slot, t, pl.ds(sub * 16, 16)] = q >> 1
            p_v[slot, t, pl.ds(sub * 16, 16)] = (q & 1) * 64
            w_v[slot, t, pl.ds(sub * 16, 16)] = jnp.where(ok, wx * wy, zero16)
    for t in range(4):
        pltpu.async_copy(feat_hbm.at[idx_v.at[slot, t]],
                         tap_v.at[slot, t], sems.at[slot])


def _gs_wait(feat_hbm, idx_v, tap_v, sems, slot):
    for t in range(4):
        pltpu.make_async_copy(feat_hbm.at[idx_v.at[slot, t]],
                              tap_v.at[slot, t], sems.at[slot]).wait()


def _gs_combine(w_v, p_v, tap_v, out_v, out_hbm, base, ci, slot):
    for sub in range(GS_CH // 16):
        wrows = [w_v[slot, t, pl.ds(sub * 16, 16)] for t in range(4)]
        prows = [p_v[slot, t, pl.ds(sub * 16, 16)] for t in range(4)]
        for p in range(16):
            pt = sub * 16 + p
            acc = [jnp.zeros((16,), jnp.float32) for _ in range(4)]
            for t in range(4):
                ws = jnp.broadcast_to(wrows[t][p], (16,))
                par = prows[t][p]
                for g in range(4):
                    v = tap_v[slot, t, pt, pl.ds(par + 16 * g, 16)]
                    acc[g] = acc[g] + ws * v
            out_v[pt, pl.ds(0, 32)] = plsc.pack(acc[0], acc[1], format=_ILV)
            out_v[pt, pl.ds(32, 32)] = plsc.pack(acc[2], acc[3], format=_ILV)
    pltpu.sync_copy(out_v, out_hbm.at[pl.ds(base + ci * GS_CH, GS_CH)])


def _grid_sample_body(feat_hbm, gath_hbm, ctx_hbm, cty_hbm, ibr_hbm, out_hbm,
                      gath_v, ctx_v, cty_v, ibr_v, idx_v, w_v, p_v,
                      tap_v, out_v, sems):
    wid = lax.axis_index("s") * 2 + lax.axis_index("c")
    base = wid * GS_PER_W
    a0 = ((base // P1) // 8) * 8
    pltpu.sync_copy(gath_hbm.at[pl.ds(2 * a0, 48)], gath_v)
    pltpu.sync_copy(ctx_hbm.at[pl.ds(a0, 24)], ctx_v)
    pltpu.sync_copy(cty_hbm.at[pl.ds(a0, 24)], cty_v)
    pltpu.sync_copy(ibr_hbm.at[pl.ds(a0, 24)], ibr_v)

    def stage(ci, slot):
        _gs_stage(gath_v, ctx_v, cty_v, ibr_v, base, a0, idx_v, w_v, p_v,
                  feat_hbm, tap_v, sems, ci, slot)

    stage(0, 0)

    def pair(j, carry):
        ci = j * 2
        stage(ci + 1, 1)
        _gs_wait(feat_hbm, idx_v, tap_v, sems, 0)
        _gs_combine(w_v, p_v, tap_v, out_v, out_hbm, base, ci, 0)
        stage(ci + 2, 0)
        _gs_wait(feat_hbm, idx_v, tap_v, sems, 1)
        _gs_combine(w_v, p_v, tap_v, out_v, out_hbm, base, ci + 1, 1)
        return carry

    lax.fori_loop(0, (GS_NCH - 1) // 2, pair, 0)
    _gs_wait(feat_hbm, idx_v, tap_v, sems, 0)
    _gs_combine(w_v, p_v, tap_v, out_v, out_hbm, base, GS_NCH - 1, 0)


def _grid_sample_sc(feat_rows, gath1024, ctx, cty, ibrow):
    k = pl.kernel(
        _grid_sample_body,
        out_type=jax.ShapeDtypeStruct((NPTS, C), jnp.bfloat16),
        mesh=plsc.VectorSubcoreMesh(core_axis_name="c", subcore_axis_name="s"),
        scratch_types=[
            pltpu.VMEM((48, 128), jnp.float32),
            pltpu.VMEM((24,), jnp.float32),
            pltpu.VMEM((24,), jnp.float32),
            pltpu.VMEM((24,), jnp.int32),
            pltpu.VMEM((2, 4, GS_CH), jnp.int32),
            pltpu.VMEM((2, 4, GS_CH), jnp.float32),
            pltpu.VMEM((2, 4, GS_CH), jnp.int32),
            pltpu.VMEM((2, 4, GS_CH, 2 * C), jnp.float32),
            pltpu.VMEM((GS_CH, C), jnp.bfloat16),
            pltpu.SemaphoreType.DMA((2,)),
        ],
        compiler_params=pltpu.CompilerParams(needs_layout_passes=False,
                                             use_tc_tiling_on_sc=False),
    )
    return k(feat_rows, gath1024, ctx, cty, ibrow)


def _conv_body(x_ref, w1_ref, b1_ref, w2_ref, b2_ref, out_ref, pad_ref):
    pad_ref[...] = jnp.zeros_like(pad_ref)
    pad_ref[1:H + 1, 1:W + 1, :] = x_ref[0].astype(jnp.bfloat16)
    for rb in range(8):
        r0 = rb * 16
        taps = []
        for dy in range(3):
            for dx in range(3):
                taps.append(
                    pad_ref[r0 + dy:r0 + dy + 16, dx:dx + W, :].reshape(16 * W, C))
        a = jnp.concatenate(taps, axis=1)
        acc = jnp.dot(a, w1_ref[...], preferred_element_type=jnp.float32)
        acc = jnp.maximum(acc + b1_ref[...], 0.0).astype(jnp.bfloat16)
        o = jnp.dot(acc, w2_ref[...], preferred_element_type=jnp.float32)
        o = o + b2_ref[...]
        o3 = o.reshape(1024, 2, C)
        out_ref[0, rb * 1024:rb * 1024 + 1024, 0:C] = o3[:, 0, :]
        out_ref[0, rb * 1024:rb * 1024 + 1024, C:2 * C] = o3[:, 1, :]


def _conv_refine(x_nhwc, w1cat, b1, w2t, b2):
    return pl.pallas_call(
        _conv_body,
        grid=(B,),
        in_specs=[
            pl.BlockSpec((1, H, W, C), lambda b: (b, 0, 0, 0)),
            pl.BlockSpec((576, 256), lambda b: (0, 0)),
            pl.BlockSpec((1, 256), lambda b: (0, 0)),
            pl.BlockSpec((256, C), lambda b: (0, 0)),
            pl.BlockSpec((1, C), lambda b: (0, 0)),
        ],
        out_specs=pl.BlockSpec((1, H * W * C // 128, 128), lambda b: (b, 0, 0)),
        out_shape=jax.ShapeDtypeStruct((B, H * W * C // 128, 128), jnp.float32),
        scratch_shapes=[pltpu.VMEM((H + 2, W + 2, C), jnp.bfloat16)],
    )(x_nhwc, w1cat, b1, w2t, b2)


def _linears_body(fp_ref, wp_ref, wf_ref, b_ref, ip4_ref, out_ref):
    pf = jax.lax.dot_general(
        fp_ref[...], wp_ref[...], (((1,), (1,)), ((), ())),
        preferred_element_type=jnp.float32)
    pf = pf.astype(jnp.bfloat16)
    out = jax.lax.dot_general(
        pf, wf_ref[...], (((1,), (1,)), ((), ())),
        preferred_element_type=jnp.float32)
    out_ref[...] = ((out + b_ref[...]) * (COARSE_STRIDE * DOWN_SAMPLE)
                    + ip4_ref[...])


def _linears(fp16, wq16, wf16, fuse_b, ip4):
    return pl.pallas_call(
        _linears_body,
        out_shape=jax.ShapeDtypeStruct((N, 256), jnp.float32),
    )(fp16, wq16, wf16, fuse_b[None, :], ip4)


def kernel(cnn_feature, wh, ct_01, ct_ind, ct_img_idx, conv1_w, conv1_b,
           conv2_w, conv2_b, trans_poly_w, trans_fuse_w, trans_fuse_b):
    mask = ct_01.reshape(-1)
    ct_ind_f = jnp.where(mask, ct_ind.reshape(-1), 0)
    img_f = jnp.where(mask, ct_img_idx.reshape(-1), 0)
    ct_x = jnp.clip(ct_ind_f % W, 0, W - 1)
    ct_y = jnp.clip(ct_ind_f // W, 0, H - 1)
    ctxf = ct_x.astype(jnp.float32)
    ctyf = ct_y.astype(jnp.float32)
    ct = jnp.stack([ctxf, ctyf], axis=1)

    cvec = jnp.arange(2 * NUM_POINT, dtype=jnp.int32) * (H * W)
    whidx = (img_f * (2 * NUM_POINT * H * W) + ct_y * W + ct_x)[:, None] + cvec[None, :]
    gath1024 = _wh_gather_sc(wh.reshape(-1), whidx.reshape(-1, 128))
    ct_tile = jnp.repeat(jnp.tile(jnp.stack([ctxf, ctyf], -1), (1, 64)), 2, axis=0)
    ipolys = gath1024 * INIT_STRIDE + ct_tile
    ip4 = (ipolys * DOWN_SAMPLE).reshape(N, 2 * NUM_POINT)

    x_nhwc = jnp.transpose(cnn_feature, (0, 2, 3, 1))
    w1cat = jnp.transpose(conv1_w, (2, 3, 1, 0)).reshape(576, 256).astype(jnp.bfloat16)
    w2t = conv2_w[:, :, 0, 0].T.astype(jnp.bfloat16)
    featrows = _conv_refine(x_nhwc, w1cat, conv1_b[None, :], w2t,
                            conv2_b[None, :]).reshape(B * H * W // 2, 2 * C)

    featpts = _grid_sample_sc(featrows, gath1024, ctxf, ctyf, img_f * (H * W))

    fp16 = featpts.reshape(N, P1 * C)
    cord = (jnp.arange(C) // 32) * 32 + jnp.where(
        jnp.arange(C) % 2 == 0, (jnp.arange(C) % 32) // 2,
        16 + (jnp.arange(C) % 32) // 2)
    wq16 = (trans_poly_w.astype(jnp.bfloat16)
            .reshape(512, C, P1).transpose(0, 2, 1)[:, :, cord]
            .reshape(512, P1 * C))
    wf16 = trans_fuse_w.astype(jnp.bfloat16)
    coarse4 = _linears(fp16, wq16, wf16, trans_fuse_b, ip4)

    return (ip4.reshape(N, NUM_POINT, 2),
            coarse4.reshape(N, NUM_POINT, 2), ct)
